# Initial kernel scaffold; baseline (speedup 1.0000x reference)
import jax
import jax.numpy as jnp
from jax.experimental import pallas as pl


def kernel(x_user, x_item, edge_index_ui, edge_index_iu, W_src_ui1, W_dst_ui1, a_src_ui1, a_dst_ui1, b_ui1, W_src_iu1, W_dst_iu1, a_src_iu1, a_dst_iu1, b_iu1, W_src_ui2, W_dst_ui2, a_src_ui2, a_dst_ui2, b_ui2, W_src_iu2, W_dst_iu2, a_src_iu2, a_dst_iu2, b_iu2, W_out_user, b_out_user, W_out_item, b_out_item):
    raise NotImplementedError("write your pallas kernel here")


# speedup vs baseline: 19.7658x; 19.7658x over previous
import functools

import jax
import jax.numpy as jnp
from jax import lax
from jax.experimental import pallas as pl
from jax.experimental.pallas import tpu as pltpu
from jax.experimental.pallas import tpu_sc as plsc

N = 10000
D = 128
E = 320000

NC = 2
NS = 16
L = 16

EPT = E // NS
K = 80
NCHUNK = EPT // K
ROWS_A = 624
RTAIL = ROWS_A - 7 * K

BLK = 2000
NB = N // BLK


def _tc_in_body(x_ref, w_ref, a_ref, h_ref, al_ref):
    x = x_ref[0]
    h_ref[0] = jnp.dot(x, w_ref[0], preferred_element_type=jnp.float32)
    al_ref[0] = jnp.dot(x, a_ref[0], preferred_element_type=jnp.float32)


def _tc_mid_body(num_ref, den_ref, b_ref, w_ref, a_ref, h_ref, al_ref):
    x = num_ref[0] / (den_ref[0] + 1e-16) + b_ref[0]
    x = jnp.maximum(x, 0.0)
    h_ref[0] = jnp.dot(x, w_ref[0], preferred_element_type=jnp.float32)
    al_ref[0] = jnp.dot(x, a_ref[0], preferred_element_type=jnp.float32)


def _tc_out_body(num_ref, den_ref, bg_ref, w_ref, bo_ref, o_ref):
    x = num_ref[0] / (den_ref[0] + 1e-16) + bg_ref[0]
    x = jnp.maximum(x, 0.0)
    o_ref[0] = (
        jnp.dot(x, w_ref[0], preferred_element_type=jnp.float32) + bo_ref[0]
    )


def _tc_in(X, W, A):
    return pl.pallas_call(
        _tc_in_body,
        grid=(2, NB),
        in_specs=[
            pl.BlockSpec((1, BLK, D), lambda s, i: (s, i, 0)),
            pl.BlockSpec((1, D, D), lambda s, i: (s, 0, 0)),
            pl.BlockSpec((1, D, 2), lambda s, i: (s, 0, 0)),
        ],
        out_specs=[
            pl.BlockSpec((1, BLK, D), lambda s, i: (s, i, 0)),
            pl.BlockSpec((1, BLK, 2), lambda s, i: (s, i, 0)),
        ],
        out_shape=[
            jax.ShapeDtypeStruct((2, N, D), jnp.float32),
            jax.ShapeDtypeStruct((2, N, 2), jnp.float32),
        ],
    )(X, W, A)


def _tc_mid(NUM, DEN, B, W, A):
    return pl.pallas_call(
        _tc_mid_body,
        grid=(2, NB),
        in_specs=[
            pl.BlockSpec((1, BLK, D), lambda s, i: (1 - s, i, 0)),
            pl.BlockSpec((1, BLK, 1), lambda s, i: (1 - s, i, 0)),
            pl.BlockSpec((1, 1, D), lambda s, i: (s, 0, 0)),
            pl.BlockSpec((1, D, D), lambda s, i: (s, 0, 0)),
            pl.BlockSpec((1, D, 2), lambda s, i: (s, 0, 0)),
        ],
        out_specs=[
            pl.BlockSpec((1, BLK, D), lambda s, i: (s, i, 0)),
            pl.BlockSpec((1, BLK, 2), lambda s, i: (s, i, 0)),
        ],
        out_shape=[
            jax.ShapeDtypeStruct((2, N, D), jnp.float32),
            jax.ShapeDtypeStruct((2, N, 2), jnp.float32),
        ],
    )(NUM, DEN, B, W, A)


def _tc_out(NUM, DEN, BG, WO, BO):
    return pl.pallas_call(
        _tc_out_body,
        grid=(2, NB),
        in_specs=[
            pl.BlockSpec((1, BLK, D), lambda s, i: (1 - s, i, 0)),
            pl.BlockSpec((1, BLK, 1), lambda s, i: (1 - s, i, 0)),
            pl.BlockSpec((1, 1, D), lambda s, i: (s, 0, 0)),
            pl.BlockSpec((1, D, D), lambda s, i: (s, 0, 0)),
            pl.BlockSpec((1, 1, D), lambda s, i: (s, 0, 0)),
        ],
        out_specs=pl.BlockSpec((1, BLK, D), lambda s, i: (s, i, 0)),
        out_shape=jax.ShapeDtypeStruct((2, N, D), jnp.float32),
    )(NUM, DEN, BG, WO, BO)


def _run_direction(slab, src_hbm, dst_hbm, h_hbm, als_hbm, ald_hbm,
                   num_out, den_out,
                   num_acc, den_fold, als_v, ald_v, src_v, dst_v, dstf_v,
                   h_in, den_up, w_buf, sem):
    sid = lax.axis_index("s")
    zf = jnp.zeros((L,), jnp.float32)
    iota16 = lax.iota(jnp.int32, L)

    pltpu.sync_copy(als_hbm, als_v)
    pltpu.sync_copy(ald_hbm, ald_v)

    def zrow_e(r, c):
        for cc in range(D // L):
            h_in[r, pl.ds(cc * L, L)] = zf
            den_up[r, pl.ds(cc * L, L)] = zf
        return c
    lax.fori_loop(0, K, zrow_e, 0)

    base = sid * ROWS_A

    def _chunk_row0(k):
        off = jnp.where((k == 7) & (sid != NS - 1), 544, k * K)
        return pl.multiple_of(base + off, 8)

    def _fill_row_idx(r):
        def body(g, c):
            src_v[pl.ds(g * L, L)] = r + g * L + iota16
            return c
        lax.fori_loop(0, K // L, body, 0)

    def zcopy_e(k, c):
        r = _chunk_row0(k)
        _fill_row_idx(r)
        pltpu.sync_copy(h_in, num_acc.at[src_v])
        return c
    lax.fori_loop(0, 8, zcopy_e, 0)

    @pl.when(sid == 0)
    def _():
        _fill_row_idx(0)
        pltpu.sync_copy(h_in, den_fold.at[src_v])

    plsc.subcore_barrier()

    ebase = sid * EPT

    def chunk(cidx, carry):
        eb = pl.multiple_of(ebase + cidx * K, 16)
        pltpu.sync_copy(src_hbm.at[pl.ds(eb, K)], src_v)
        pltpu.sync_copy(dst_hbm.at[pl.ds(eb, K)], dst_v)
        pltpu.async_copy(h_hbm.at[src_v], h_in, sem).wait()

        def group(g, c2):
            s16 = src_v[pl.ds(g * L, L)]
            d16 = dst_v[pl.ds(g * L, L)]
            e16 = plsc.load_gather(als_v, [s16]) + plsc.load_gather(ald_v, [d16])
            e16 = jnp.where(e16 > 0, e16, 0.2 * e16)
            w16 = jnp.exp(e16)
            row16 = g * L + iota16
            w_buf[pl.ds(g * L, L)] = w16
            plsc.store_scatter(den_up, [row16, d16 & 127], w16)
            dstf_v[pl.ds(g * L, L)] = lax.shift_right_logical(d16, 7)
            for j in range(L):
                e = g * L + j
                sp = plsc.load_gather(w_buf, [jnp.full((L,), e, jnp.int32)])
                for cc in range(D // L):
                    h_in[e, pl.ds(cc * L, L)] = h_in[e, pl.ds(cc * L, L)] * sp
            return c2
        lax.fori_loop(0, K // L, group, 0)

        pltpu.sync_copy(h_in, num_acc.at[dst_v], add=True)
        pltpu.sync_copy(den_up, den_fold.at[dstf_v], add=True)

        def clean(g, c2):
            d16 = dst_v[pl.ds(g * L, L)]
            plsc.store_scatter(den_up, [g * L + iota16, d16 & 127], zf)
            return c2
        lax.fori_loop(0, K // L, clean, 0)
        return carry
    lax.fori_loop(0, NCHUNK, chunk, 0)

    plsc.subcore_barrier()

    def rcopy_e(k, c):
        r = _chunk_row0(k)
        _fill_row_idx(r)
        pltpu.async_copy(num_acc.at[src_v], h_in, sem).wait()
        pltpu.sync_copy(h_in, num_out.at[slab, pl.ds(r, K)])
        return c
    lax.fori_loop(0, 8, rcopy_e, 0)

    @pl.when(sid == 0)
    def _():
        _fill_row_idx(0)
        pltpu.async_copy(den_fold.at[src_v], h_in, sem).wait()
        pltpu.sync_copy(h_in, den_out.at[slab])
    return


@functools.partial(
    pl.kernel,
    out_type=(
        jax.ShapeDtypeStruct((2, N, D), jnp.float32),
        jax.ShapeDtypeStruct((2, K, D), jnp.float32),
    ),
    mesh=plsc.VectorSubcoreMesh(
        core_axis_name="c", subcore_axis_name="s", num_cores=NC,
        num_subcores=NS,
    ),
    compiler_params=pltpu.CompilerParams(needs_layout_passes=False),
    scratch_types=[
        pltpu.VMEM_SHARED((N, D), jnp.float32),
        pltpu.VMEM_SHARED((K, D), jnp.float32),
        pltpu.VMEM((N,), jnp.float32),
        pltpu.VMEM((N,), jnp.float32),
        pltpu.VMEM((K,), jnp.int32),
        pltpu.VMEM((K,), jnp.int32),
        pltpu.VMEM((K,), jnp.int32),
        pltpu.VMEM((K, D), jnp.float32),
        pltpu.VMEM((K, D), jnp.float32),
        pltpu.VMEM((K,), jnp.float32),
        pltpu.SemaphoreType.DMA,
    ],
)
def _sc_edge_kernel(src_ui, dst_ui, src_iu, dst_iu, h_u, h_i,
                    als_ui, ald_ui, als_iu, ald_iu,
                    num_out, den_out,
                    num_acc, den_fold, als_v, ald_v, src_v, dst_v, dstf_v,
                    h_in, den_up, w_buf, sem):
    _sc_body(src_ui, dst_ui, src_iu, dst_iu, h_u, h_i,
             als_ui, ald_ui, als_iu, ald_iu,
             num_out, den_out,
             num_acc, den_fold, als_v, ald_v, src_v, dst_v, dstf_v,
             h_in, den_up, w_buf, sem)


def _sc_body(src_ui, dst_ui, src_iu, dst_iu, h_u, h_i,
             als_ui, ald_ui, als_iu, ald_iu,
             num_out, den_out,
             num_acc, den_fold, als_v, ald_v, src_v, dst_v, dstf_v,
             h_in, den_up, w_buf, sem):
    cid = lax.axis_index("c")

    @pl.when(cid == 0)
    def _():
        _run_direction(0, src_ui, dst_ui, h_u, als_ui, ald_ui,
                       num_out, den_out, num_acc, den_fold, als_v, ald_v,
                       src_v, dst_v, dstf_v, h_in, den_up, w_buf, sem)

    @pl.when(cid == 1)
    def _():
        _run_direction(1, src_iu, dst_iu, h_i, als_iu, ald_iu,
                       num_out, den_out, num_acc, den_fold, als_v, ald_v,
                       src_v, dst_v, dstf_v, h_in, den_up, w_buf, sem)


def _sc_pass(src_ui, dst_ui, src_iu, dst_iu, H, AL):
    return _sc_edge_kernel(
        src_ui, dst_ui, src_iu, dst_iu, H[0], H[1],
        AL[0, :, 0], AL[1, :, 1], AL[1, :, 0], AL[0, :, 1])


def _den_cols(den):
    return den.reshape(2, K * D)[:, :N, None]


def kernel(x_user, x_item, edge_index_ui, edge_index_iu,
           W_src_ui1, W_dst_ui1, a_src_ui1, a_dst_ui1, b_ui1,
           W_src_iu1, W_dst_iu1, a_src_iu1, a_dst_iu1, b_iu1,
           W_src_ui2, W_dst_ui2, a_src_ui2, a_dst_ui2, b_ui2,
           W_src_iu2, W_dst_iu2, a_src_iu2, a_dst_iu2, b_iu2,
           W_out_user, b_out_user, W_out_item, b_out_item):
    X = jnp.stack([x_user, x_item])
    src_ui, dst_ui = edge_index_ui[0], edge_index_ui[1]
    src_iu, dst_iu = edge_index_iu[0], edge_index_iu[1]

    W1 = jnp.stack([W_src_ui1, W_src_iu1])
    A1 = jnp.stack([
        jnp.stack([W_src_ui1 @ a_src_ui1, W_dst_iu1 @ a_dst_iu1], axis=1),
        jnp.stack([W_src_iu1 @ a_src_iu1, W_dst_ui1 @ a_dst_ui1], axis=1),
    ])
    H1, AL1 = _tc_in(X, W1, A1)
    num1, den1 = _sc_pass(src_ui, dst_ui, src_iu, dst_iu, H1, AL1)

    B1 = jnp.stack([b_iu1, b_ui1])[:, None, :]
    W2 = jnp.stack([W_src_ui2, W_src_iu2])
    A2 = jnp.stack([
        jnp.stack([W_src_ui2 @ a_src_ui2, W_dst_iu2 @ a_dst_iu2], axis=1),
        jnp.stack([W_src_iu2 @ a_src_iu2, W_dst_ui2 @ a_dst_ui2], axis=1),
    ])
    H2, AL2 = _tc_mid(num1, _den_cols(den1), B1, W2, A2)
    num2, den2 = _sc_pass(src_ui, dst_ui, src_iu, dst_iu, H2, AL2)

    BG = jnp.stack([b_iu2, b_ui2])[:, None, :]
    WO = jnp.stack([W_out_user, W_out_item])
    BO = jnp.stack([b_out_user, b_out_item])[:, None, :]
    OUT = _tc_out(num2, _den_cols(den2), BG, WO, BO)
    return OUT[0], OUT[1]


# speedup vs baseline: 22.3759x; 1.1321x over previous
import functools

import jax
import jax.numpy as jnp
from jax import lax
from jax.experimental import pallas as pl
from jax.experimental.pallas import tpu as pltpu
from jax.experimental.pallas import tpu_sc as plsc

N = 10000
D = 128
E = 320000

NC = 2
NS = 16
L = 16

EPT = E // NS
K = 80
NCHUNK = EPT // K
ROWS_A = 624
RTAIL = ROWS_A - 7 * K

BLK = 2000
NB = N // BLK


def _tc_in_body(x_ref, w_ref, a_ref, h_ref, al_ref):
    x = x_ref[0]
    h_ref[0] = jnp.dot(x, w_ref[0], preferred_element_type=jnp.float32)
    al_ref[0] = jnp.dot(x, a_ref[0], preferred_element_type=jnp.float32)


def _tc_mid_body(num_ref, den_ref, b_ref, w_ref, a_ref, h_ref, al_ref):
    x = num_ref[0] / (den_ref[0] + 1e-16) + b_ref[0]
    x = jnp.maximum(x, 0.0)
    h_ref[0] = jnp.dot(x, w_ref[0], preferred_element_type=jnp.float32)
    al_ref[0] = jnp.dot(x, a_ref[0], preferred_element_type=jnp.float32)


def _tc_out_body(num_ref, den_ref, bg_ref, w_ref, bo_ref, o_ref):
    x = num_ref[0] / (den_ref[0] + 1e-16) + bg_ref[0]
    x = jnp.maximum(x, 0.0)
    o_ref[0] = (
        jnp.dot(x, w_ref[0], preferred_element_type=jnp.float32) + bo_ref[0]
    )


def _tc_in(X, W, A):
    return pl.pallas_call(
        _tc_in_body,
        grid=(2, NB),
        in_specs=[
            pl.BlockSpec((1, BLK, D), lambda s, i: (s, i, 0)),
            pl.BlockSpec((1, D, D), lambda s, i: (s, 0, 0)),
            pl.BlockSpec((1, D, 2), lambda s, i: (s, 0, 0)),
        ],
        out_specs=[
            pl.BlockSpec((1, BLK, D), lambda s, i: (s, i, 0)),
            pl.BlockSpec((1, BLK, 2), lambda s, i: (s, i, 0)),
        ],
        out_shape=[
            jax.ShapeDtypeStruct((2, N, D), jnp.float32),
            jax.ShapeDtypeStruct((2, N, 2), jnp.float32),
        ],
    )(X, W, A)


def _tc_mid(NUM, DEN, B, W, A):
    return pl.pallas_call(
        _tc_mid_body,
        grid=(2, NB),
        in_specs=[
            pl.BlockSpec((1, BLK, D), lambda s, i: (1 - s, i, 0)),
            pl.BlockSpec((1, BLK, 1), lambda s, i: (1 - s, i, 0)),
            pl.BlockSpec((1, 1, D), lambda s, i: (s, 0, 0)),
            pl.BlockSpec((1, D, D), lambda s, i: (s, 0, 0)),
            pl.BlockSpec((1, D, 2), lambda s, i: (s, 0, 0)),
        ],
        out_specs=[
            pl.BlockSpec((1, BLK, D), lambda s, i: (s, i, 0)),
            pl.BlockSpec((1, BLK, 2), lambda s, i: (s, i, 0)),
        ],
        out_shape=[
            jax.ShapeDtypeStruct((2, N, D), jnp.float32),
            jax.ShapeDtypeStruct((2, N, 2), jnp.float32),
        ],
    )(NUM, DEN, B, W, A)


def _tc_out(NUM, DEN, BG, WO, BO):
    return pl.pallas_call(
        _tc_out_body,
        grid=(2, NB),
        in_specs=[
            pl.BlockSpec((1, BLK, D), lambda s, i: (1 - s, i, 0)),
            pl.BlockSpec((1, BLK, 1), lambda s, i: (1 - s, i, 0)),
            pl.BlockSpec((1, 1, D), lambda s, i: (s, 0, 0)),
            pl.BlockSpec((1, D, D), lambda s, i: (s, 0, 0)),
            pl.BlockSpec((1, 1, D), lambda s, i: (s, 0, 0)),
        ],
        out_specs=pl.BlockSpec((1, BLK, D), lambda s, i: (s, i, 0)),
        out_shape=jax.ShapeDtypeStruct((2, N, D), jnp.float32),
    )(NUM, DEN, BG, WO, BO)


def _run_direction(slab, src_hbm, dst_hbm, h_hbm, als_hbm, ald_hbm,
                   num_out, den_out,
                   num_acc, den_fold, als_v, ald_v, src_v, dst_v, dstf_v,
                   h_in, den_up, w_buf, sem, sem_a, sem_b):
    sid = lax.axis_index("s")
    zf = jnp.zeros((L,), jnp.float32)
    iota16 = lax.iota(jnp.int32, L)

    pltpu.sync_copy(als_hbm, als_v)
    pltpu.sync_copy(ald_hbm, ald_v)

    def zrow_e(r, c):
        for cc in range(D // L):
            h_in[r, pl.ds(cc * L, L)] = zf
            den_up[r, pl.ds(cc * L, L)] = zf
        return c
    lax.fori_loop(0, K, zrow_e, 0)

    base = sid * ROWS_A

    def _chunk_row0(k):
        off = jnp.where((k == 7) & (sid != NS - 1), 544, k * K)
        return pl.multiple_of(base + off, 8)

    def _fill_row_idx(r):
        def body(g, c):
            src_v[pl.ds(g * L, L)] = r + g * L + iota16
            return c
        lax.fori_loop(0, K // L, body, 0)

    def zcopy_e(k, c):
        r = _chunk_row0(k)
        _fill_row_idx(r)
        pltpu.sync_copy(h_in, num_acc.at[src_v])
        return c
    lax.fori_loop(0, 8, zcopy_e, 0)

    @pl.when(sid == 0)
    def _():
        _fill_row_idx(0)
        pltpu.sync_copy(h_in, den_fold.at[src_v])

    plsc.subcore_barrier()

    ebase = sid * EPT

    def chunk(cidx, carry):
        eb = pl.multiple_of(ebase + cidx * K, 16)
        ca = pltpu.async_copy(src_hbm.at[pl.ds(eb, K)], src_v, sem_a)
        cb = pltpu.async_copy(dst_hbm.at[pl.ds(eb, K)], dst_v, sem_b)
        ca.wait()
        cb.wait()
        pltpu.async_copy(h_hbm.at[src_v], h_in, sem).wait()

        def group(g, c2):
            s16 = src_v[pl.ds(g * L, L)]
            d16 = dst_v[pl.ds(g * L, L)]
            e16 = plsc.load_gather(als_v, [s16]) + plsc.load_gather(ald_v, [d16])
            e16 = jnp.where(e16 > 0, e16, 0.2 * e16)
            w16 = jnp.exp(e16)
            row16 = g * L + iota16
            w_buf[pl.ds(g * L, L)] = w16
            plsc.store_scatter(den_up, [row16, d16 & 127], w16)
            dstf_v[pl.ds(g * L, L)] = lax.shift_right_logical(d16, 7)
            for j in range(L):
                e = g * L + j
                sp = plsc.load_gather(w_buf, [jnp.full((L,), e, jnp.int32)])
                for cc in range(D // L):
                    h_in[e, pl.ds(cc * L, L)] = h_in[e, pl.ds(cc * L, L)] * sp
            return c2
        lax.fori_loop(0, K // L, group, 0)

        sa = pltpu.async_copy(h_in, num_acc.at[dst_v], sem_a, add=True)
        sb = pltpu.async_copy(den_up, den_fold.at[dstf_v], sem_b, add=True)
        sa.wait()
        sb.wait()

        def clean(g, c2):
            d16 = dst_v[pl.ds(g * L, L)]
            plsc.store_scatter(den_up, [g * L + iota16, d16 & 127], zf)
            return c2
        lax.fori_loop(0, K // L, clean, 0)
        return carry
    lax.fori_loop(0, NCHUNK, chunk, 0)

    plsc.subcore_barrier()

    def rcopy_e(k, c):
        r = _chunk_row0(k)
        _fill_row_idx(r)
        pltpu.async_copy(num_acc.at[src_v], h_in, sem).wait()
        pltpu.sync_copy(h_in, num_out.at[slab, pl.ds(r, K)])
        return c
    lax.fori_loop(0, 8, rcopy_e, 0)

    @pl.when(sid == 0)
    def _():
        _fill_row_idx(0)
        pltpu.async_copy(den_fold.at[src_v], h_in, sem).wait()
        pltpu.sync_copy(h_in, den_out.at[slab])
    return


@functools.partial(
    pl.kernel,
    out_type=(
        jax.ShapeDtypeStruct((2, N, D), jnp.float32),
        jax.ShapeDtypeStruct((2, K, D), jnp.float32),
    ),
    mesh=plsc.VectorSubcoreMesh(
        core_axis_name="c", subcore_axis_name="s", num_cores=NC,
        num_subcores=NS,
    ),
    compiler_params=pltpu.CompilerParams(needs_layout_passes=False),
    scratch_types=[
        pltpu.VMEM_SHARED((N, D), jnp.float32),
        pltpu.VMEM_SHARED((K, D), jnp.float32),
        pltpu.VMEM((N,), jnp.float32),
        pltpu.VMEM((N,), jnp.float32),
        pltpu.VMEM((K,), jnp.int32),
        pltpu.VMEM((K,), jnp.int32),
        pltpu.VMEM((K,), jnp.int32),
        pltpu.VMEM((K, D), jnp.float32),
        pltpu.VMEM((K, D), jnp.float32),
        pltpu.VMEM((K,), jnp.float32),
        pltpu.SemaphoreType.DMA,
        pltpu.SemaphoreType.DMA,
        pltpu.SemaphoreType.DMA,
    ],
)
def _sc_edge_kernel(src_ui, dst_ui, src_iu, dst_iu, h_u, h_i,
                    als_ui, ald_ui, als_iu, ald_iu,
                    num_out, den_out,
                    num_acc, den_fold, als_v, ald_v, src_v, dst_v, dstf_v,
                    h_in, den_up, w_buf, sem, sem_a, sem_b):
    _sc_body(src_ui, dst_ui, src_iu, dst_iu, h_u, h_i,
             als_ui, ald_ui, als_iu, ald_iu,
             num_out, den_out,
             num_acc, den_fold, als_v, ald_v, src_v, dst_v, dstf_v,
             h_in, den_up, w_buf, sem, sem_a, sem_b)


def _sc_body(src_ui, dst_ui, src_iu, dst_iu, h_u, h_i,
             als_ui, ald_ui, als_iu, ald_iu,
             num_out, den_out,
             num_acc, den_fold, als_v, ald_v, src_v, dst_v, dstf_v,
             h_in, den_up, w_buf, sem, sem_a, sem_b):
    cid = lax.axis_index("c")

    @pl.when(cid == 0)
    def _():
        _run_direction(0, src_ui, dst_ui, h_u, als_ui, ald_ui,
                       num_out, den_out, num_acc, den_fold, als_v, ald_v,
                       src_v, dst_v, dstf_v, h_in, den_up, w_buf, sem, sem_a, sem_b)

    @pl.when(cid == 1)
    def _():
        _run_direction(1, src_iu, dst_iu, h_i, als_iu, ald_iu,
                       num_out, den_out, num_acc, den_fold, als_v, ald_v,
                       src_v, dst_v, dstf_v, h_in, den_up, w_buf, sem, sem_a, sem_b)


def _sc_pass(src_ui, dst_ui, src_iu, dst_iu, H, AL):
    return _sc_edge_kernel(
        src_ui, dst_ui, src_iu, dst_iu, H[0], H[1],
        AL[0, :, 0], AL[1, :, 1], AL[1, :, 0], AL[0, :, 1])


def _den_cols(den):
    return den.reshape(2, K * D)[:, :N, None]


def kernel(x_user, x_item, edge_index_ui, edge_index_iu,
           W_src_ui1, W_dst_ui1, a_src_ui1, a_dst_ui1, b_ui1,
           W_src_iu1, W_dst_iu1, a_src_iu1, a_dst_iu1, b_iu1,
           W_src_ui2, W_dst_ui2, a_src_ui2, a_dst_ui2, b_ui2,
           W_src_iu2, W_dst_iu2, a_src_iu2, a_dst_iu2, b_iu2,
           W_out_user, b_out_user, W_out_item, b_out_item):
    X = jnp.stack([x_user, x_item])
    src_ui, dst_ui = edge_index_ui[0], edge_index_ui[1]
    src_iu, dst_iu = edge_index_iu[0], edge_index_iu[1]

    W1 = jnp.stack([W_src_ui1, W_src_iu1])
    A1 = jnp.stack([
        jnp.stack([W_src_ui1 @ a_src_ui1, W_dst_iu1 @ a_dst_iu1], axis=1),
        jnp.stack([W_src_iu1 @ a_src_iu1, W_dst_ui1 @ a_dst_ui1], axis=1),
    ])
    H1, AL1 = _tc_in(X, W1, A1)
    num1, den1 = _sc_pass(src_ui, dst_ui, src_iu, dst_iu, H1, AL1)

    B1 = jnp.stack([b_iu1, b_ui1])[:, None, :]
    W2 = jnp.stack([W_src_ui2, W_src_iu2])
    A2 = jnp.stack([
        jnp.stack([W_src_ui2 @ a_src_ui2, W_dst_iu2 @ a_dst_iu2], axis=1),
        jnp.stack([W_src_iu2 @ a_src_iu2, W_dst_ui2 @ a_dst_ui2], axis=1),
    ])
    H2, AL2 = _tc_mid(num1, _den_cols(den1), B1, W2, A2)
    num2, den2 = _sc_pass(src_ui, dst_ui, src_iu, dst_iu, H2, AL2)

    BG = jnp.stack([b_iu2, b_ui2])[:, None, :]
    WO = jnp.stack([W_out_user, W_out_item])
    BO = jnp.stack([b_out_user, b_out_item])[:, None, :]
    OUT = _tc_out(num2, _den_cols(den2), BG, WO, BO)
    return OUT[0], OUT[1]


# speedup vs baseline: 26.1544x; 1.1689x over previous
import functools

import jax
import jax.numpy as jnp
from jax import lax
from jax.experimental import pallas as pl
from jax.experimental.pallas import tpu as pltpu
from jax.experimental.pallas import tpu_sc as plsc

N = 10000
D = 128
E = 320000

NC = 2
NS = 16
L = 16

EPT = E // NS
K = 80
NCHUNK = EPT // K
ROWS_A = 624
RTAIL = ROWS_A - 7 * K

BLK = 2000
NB = N // BLK


def _tc_in_body(x_ref, w_ref, a_ref, h_ref, al_ref):
    x = x_ref[0]
    h_ref[0] = jnp.dot(x, w_ref[0], preferred_element_type=jnp.float32)
    al_ref[0] = jnp.dot(x, a_ref[0], preferred_element_type=jnp.float32)


def _tc_mid_body(num_ref, den_ref, b_ref, w_ref, a_ref, h_ref, al_ref):
    x = num_ref[0] / (den_ref[0] + 1e-16) + b_ref[0]
    x = jnp.maximum(x, 0.0)
    h_ref[0] = jnp.dot(x, w_ref[0], preferred_element_type=jnp.float32)
    al_ref[0] = jnp.dot(x, a_ref[0], preferred_element_type=jnp.float32)


def _tc_out_body(num_ref, den_ref, bg_ref, w_ref, bo_ref, o_ref):
    x = num_ref[0] / (den_ref[0] + 1e-16) + bg_ref[0]
    x = jnp.maximum(x, 0.0)
    o_ref[0] = (
        jnp.dot(x, w_ref[0], preferred_element_type=jnp.float32) + bo_ref[0]
    )


def _tc_in(X, W, A):
    return pl.pallas_call(
        _tc_in_body,
        grid=(2, NB),
        in_specs=[
            pl.BlockSpec((1, BLK, D), lambda s, i: (s, i, 0)),
            pl.BlockSpec((1, D, D), lambda s, i: (s, 0, 0)),
            pl.BlockSpec((1, D, 2), lambda s, i: (s, 0, 0)),
        ],
        out_specs=[
            pl.BlockSpec((1, BLK, D), lambda s, i: (s, i, 0)),
            pl.BlockSpec((1, BLK, 2), lambda s, i: (s, i, 0)),
        ],
        out_shape=[
            jax.ShapeDtypeStruct((2, N, D), jnp.float32),
            jax.ShapeDtypeStruct((2, N, 2), jnp.float32),
        ],
    )(X, W, A)


def _tc_mid(NUM, DEN, B, W, A):
    return pl.pallas_call(
        _tc_mid_body,
        grid=(2, NB),
        in_specs=[
            pl.BlockSpec((1, BLK, D), lambda s, i: (1 - s, i, 0)),
            pl.BlockSpec((1, BLK, 1), lambda s, i: (1 - s, i, 0)),
            pl.BlockSpec((1, 1, D), lambda s, i: (s, 0, 0)),
            pl.BlockSpec((1, D, D), lambda s, i: (s, 0, 0)),
            pl.BlockSpec((1, D, 2), lambda s, i: (s, 0, 0)),
        ],
        out_specs=[
            pl.BlockSpec((1, BLK, D), lambda s, i: (s, i, 0)),
            pl.BlockSpec((1, BLK, 2), lambda s, i: (s, i, 0)),
        ],
        out_shape=[
            jax.ShapeDtypeStruct((2, N, D), jnp.float32),
            jax.ShapeDtypeStruct((2, N, 2), jnp.float32),
        ],
    )(NUM, DEN, B, W, A)


def _tc_out(NUM, DEN, BG, WO, BO):
    return pl.pallas_call(
        _tc_out_body,
        grid=(2, NB),
        in_specs=[
            pl.BlockSpec((1, BLK, D), lambda s, i: (1 - s, i, 0)),
            pl.BlockSpec((1, BLK, 1), lambda s, i: (1 - s, i, 0)),
            pl.BlockSpec((1, 1, D), lambda s, i: (s, 0, 0)),
            pl.BlockSpec((1, D, D), lambda s, i: (s, 0, 0)),
            pl.BlockSpec((1, 1, D), lambda s, i: (s, 0, 0)),
        ],
        out_specs=pl.BlockSpec((1, BLK, D), lambda s, i: (s, i, 0)),
        out_shape=jax.ShapeDtypeStruct((2, N, D), jnp.float32),
    )(NUM, DEN, BG, WO, BO)


def _run_direction(slab, src_hbm, dst_hbm, h_hbm, als_hbm, ald_hbm,
                   num_out, den_out,
                   num_acc, den_fold, als_v, ald_v, src_v, dst_v, dstf_v,
                   h_in, den_up, w_buf, sem, sem_a, sem_b):
    sid = lax.axis_index("s")
    zf = jnp.zeros((L,), jnp.float32)
    iota16 = lax.iota(jnp.int32, L)

    pltpu.sync_copy(als_hbm, als_v)
    pltpu.sync_copy(ald_hbm, ald_v)

    def zrow_e(r, c):
        for cc in range(D // L):
            h_in[r, pl.ds(cc * L, L)] = zf
            den_up[r, pl.ds(cc * L, L)] = zf
        return c
    lax.fori_loop(0, K, zrow_e, 0)

    base = sid * ROWS_A

    def _chunk_row0(k):
        off = jnp.where((k == 7) & (sid != NS - 1), 544, k * K)
        return pl.multiple_of(base + off, 8)

    def _fill_row_idx(r):
        def body(g, c):
            src_v[0, pl.ds(g * L, L)] = r + g * L + iota16
            return c
        lax.fori_loop(0, K // L, body, 0)

    def zcopy_e(k, c):
        r = _chunk_row0(k)
        _fill_row_idx(r)
        pltpu.sync_copy(h_in, num_acc.at[src_v.at[0]])
        return c
    lax.fori_loop(0, 8, zcopy_e, 0)

    @pl.when(sid == 0)
    def _():
        _fill_row_idx(0)
        pltpu.sync_copy(h_in, den_fold.at[src_v.at[0]])

    plsc.subcore_barrier()

    ebase = sid * EPT

    eb0 = pl.multiple_of(ebase, 16)
    pltpu.sync_copy(src_hbm.at[pl.ds(eb0, K)], src_v.at[0])
    pltpu.sync_copy(dst_hbm.at[pl.ds(eb0, K)], dst_v.at[0])

    def chunk(cidx, carry):
        b = cidx & 1
        gd = pltpu.async_copy(h_hbm.at[src_v.at[b]], h_in, sem)
        ebn = pl.multiple_of(
            ebase + jnp.minimum(cidx + 1, NCHUNK - 1) * K, 16)
        ca = pltpu.async_copy(src_hbm.at[pl.ds(ebn, K)], src_v.at[1 - b],
                              sem_a)
        cb = pltpu.async_copy(dst_hbm.at[pl.ds(ebn, K)], dst_v.at[1 - b],
                              sem_b)

        def wgroup(g, c2):
            s16 = src_v[b, pl.ds(g * L, L)]
            d16 = dst_v[b, pl.ds(g * L, L)]
            e16 = plsc.load_gather(als_v, [s16]) + plsc.load_gather(ald_v, [d16])
            e16 = jnp.where(e16 > 0, e16, 0.2 * e16)
            w16 = jnp.exp(e16)
            w_buf[pl.ds(g * L, L)] = w16
            plsc.store_scatter(den_up, [g * L + iota16, d16 & 127], w16)
            dstf_v[pl.ds(g * L, L)] = lax.shift_right_logical(d16, 7)
            return c2
        lax.fori_loop(0, K // L, wgroup, 0)
        gd.wait()

        def sgroup(g, c2):
            for j in range(L):
                e = g * L + j
                sp = plsc.load_gather(w_buf, [jnp.full((L,), e, jnp.int32)])
                for cc in range(D // L):
                    h_in[e, pl.ds(cc * L, L)] = h_in[e, pl.ds(cc * L, L)] * sp
            return c2
        lax.fori_loop(0, K // L, sgroup, 0)
        ca.wait()
        cb.wait()

        sa = pltpu.async_copy(h_in, num_acc.at[dst_v.at[b]], sem_a, add=True)
        sb = pltpu.async_copy(den_up, den_fold.at[dstf_v], sem_b, add=True)
        sb.wait()

        def clean(g, c2):
            d16 = dst_v[b, pl.ds(g * L, L)]
            plsc.store_scatter(den_up, [g * L + iota16, d16 & 127], zf)
            return c2
        lax.fori_loop(0, K // L, clean, 0)
        sa.wait()
        return carry
    lax.fori_loop(0, NCHUNK, chunk, 0)

    plsc.subcore_barrier()

    def rcopy_e(k, c):
        r = _chunk_row0(k)
        _fill_row_idx(r)
        pltpu.async_copy(num_acc.at[src_v.at[0]], h_in, sem).wait()
        pltpu.sync_copy(h_in, num_out.at[slab, pl.ds(r, K)])
        return c
    lax.fori_loop(0, 8, rcopy_e, 0)

    @pl.when(sid == 0)
    def _():
        _fill_row_idx(0)
        pltpu.async_copy(den_fold.at[src_v.at[0]], h_in, sem).wait()
        pltpu.sync_copy(h_in, den_out.at[slab])
    return


@functools.partial(
    pl.kernel,
    out_type=(
        jax.ShapeDtypeStruct((2, N, D), jnp.float32),
        jax.ShapeDtypeStruct((2, K, D), jnp.float32),
    ),
    mesh=plsc.VectorSubcoreMesh(
        core_axis_name="c", subcore_axis_name="s", num_cores=NC,
        num_subcores=NS,
    ),
    compiler_params=pltpu.CompilerParams(needs_layout_passes=False),
    scratch_types=[
        pltpu.VMEM_SHARED((N, D), jnp.float32),
        pltpu.VMEM_SHARED((K, D), jnp.float32),
        pltpu.VMEM((N,), jnp.float32),
        pltpu.VMEM((N,), jnp.float32),
        pltpu.VMEM((2, K), jnp.int32),
        pltpu.VMEM((2, K), jnp.int32),
        pltpu.VMEM((K,), jnp.int32),
        pltpu.VMEM((K, D), jnp.float32),
        pltpu.VMEM((K, D), jnp.float32),
        pltpu.VMEM((K,), jnp.float32),
        pltpu.SemaphoreType.DMA,
        pltpu.SemaphoreType.DMA,
        pltpu.SemaphoreType.DMA,
    ],
)
def _sc_edge_kernel(src_ui, dst_ui, src_iu, dst_iu, h_u, h_i,
                    als_ui, ald_ui, als_iu, ald_iu,
                    num_out, den_out,
                    num_acc, den_fold, als_v, ald_v, src_v, dst_v, dstf_v,
                    h_in, den_up, w_buf, sem, sem_a, sem_b):
    _sc_body(src_ui, dst_ui, src_iu, dst_iu, h_u, h_i,
             als_ui, ald_ui, als_iu, ald_iu,
             num_out, den_out,
             num_acc, den_fold, als_v, ald_v, src_v, dst_v, dstf_v,
             h_in, den_up, w_buf, sem, sem_a, sem_b)


def _sc_body(src_ui, dst_ui, src_iu, dst_iu, h_u, h_i,
             als_ui, ald_ui, als_iu, ald_iu,
             num_out, den_out,
             num_acc, den_fold, als_v, ald_v, src_v, dst_v, dstf_v,
             h_in, den_up, w_buf, sem, sem_a, sem_b):
    cid = lax.axis_index("c")

    @pl.when(cid == 0)
    def _():
        _run_direction(0, src_ui, dst_ui, h_u, als_ui, ald_ui,
                       num_out, den_out, num_acc, den_fold, als_v, ald_v,
                       src_v, dst_v, dstf_v, h_in, den_up, w_buf, sem, sem_a, sem_b)

    @pl.when(cid == 1)
    def _():
        _run_direction(1, src_iu, dst_iu, h_i, als_iu, ald_iu,
                       num_out, den_out, num_acc, den_fold, als_v, ald_v,
                       src_v, dst_v, dstf_v, h_in, den_up, w_buf, sem, sem_a, sem_b)


def _sc_pass(src_ui, dst_ui, src_iu, dst_iu, H, AL):
    return _sc_edge_kernel(
        src_ui, dst_ui, src_iu, dst_iu, H[0], H[1],
        AL[0, :, 0], AL[1, :, 1], AL[1, :, 0], AL[0, :, 1])


def _den_cols(den):
    return den.reshape(2, K * D)[:, :N, None]


def kernel(x_user, x_item, edge_index_ui, edge_index_iu,
           W_src_ui1, W_dst_ui1, a_src_ui1, a_dst_ui1, b_ui1,
           W_src_iu1, W_dst_iu1, a_src_iu1, a_dst_iu1, b_iu1,
           W_src_ui2, W_dst_ui2, a_src_ui2, a_dst_ui2, b_ui2,
           W_src_iu2, W_dst_iu2, a_src_iu2, a_dst_iu2, b_iu2,
           W_out_user, b_out_user, W_out_item, b_out_item):
    X = jnp.stack([x_user, x_item])
    src_ui, dst_ui = edge_index_ui[0], edge_index_ui[1]
    src_iu, dst_iu = edge_index_iu[0], edge_index_iu[1]

    W1 = jnp.stack([W_src_ui1, W_src_iu1])
    A1 = jnp.stack([
        jnp.stack([W_src_ui1 @ a_src_ui1, W_dst_iu1 @ a_dst_iu1], axis=1),
        jnp.stack([W_src_iu1 @ a_src_iu1, W_dst_ui1 @ a_dst_ui1], axis=1),
    ])
    H1, AL1 = _tc_in(X, W1, A1)
    num1, den1 = _sc_pass(src_ui, dst_ui, src_iu, dst_iu, H1, AL1)

    B1 = jnp.stack([b_iu1, b_ui1])[:, None, :]
    W2 = jnp.stack([W_src_ui2, W_src_iu2])
    A2 = jnp.stack([
        jnp.stack([W_src_ui2 @ a_src_ui2, W_dst_iu2 @ a_dst_iu2], axis=1),
        jnp.stack([W_src_iu2 @ a_src_iu2, W_dst_ui2 @ a_dst_ui2], axis=1),
    ])
    H2, AL2 = _tc_mid(num1, _den_cols(den1), B1, W2, A2)
    num2, den2 = _sc_pass(src_ui, dst_ui, src_iu, dst_iu, H2, AL2)

    BG = jnp.stack([b_iu2, b_ui2])[:, None, :]
    WO = jnp.stack([W_out_user, W_out_item])
    BO = jnp.stack([b_out_user, b_out_item])[:, None, :]
    OUT = _tc_out(num2, _den_cols(den2), BG, WO, BO)
    return OUT[0], OUT[1]


# speedup vs baseline: 30.0834x; 1.1502x over previous
import functools

import jax
import jax.numpy as jnp
from jax import lax
from jax.experimental import pallas as pl
from jax.experimental.pallas import tpu as pltpu
from jax.experimental.pallas import tpu_sc as plsc

N = 10000
D = 128
E = 320000

NC = 2
NS = 16
L = 16

EPT = E // NS
K = 80
NCHUNK = EPT // K
ROWS_A = 624
RTAIL = ROWS_A - 7 * K

BLK = 2000
NB = N // BLK


def _tc_in_body(x_ref, w_ref, a_ref, h_ref, al_ref):
    x = x_ref[0]
    h_ref[0] = jnp.dot(x, w_ref[0], preferred_element_type=jnp.float32)
    al_ref[0] = jnp.dot(x, a_ref[0], preferred_element_type=jnp.float32)


def _tc_mid_body(num_ref, den_ref, b_ref, w_ref, a_ref, h_ref, al_ref):
    x = num_ref[0] / (den_ref[0] + 1e-16) + b_ref[0]
    x = jnp.maximum(x, 0.0)
    h_ref[0] = jnp.dot(x, w_ref[0], preferred_element_type=jnp.float32)
    al_ref[0] = jnp.dot(x, a_ref[0], preferred_element_type=jnp.float32)


def _tc_out_body(num_ref, den_ref, bg_ref, w_ref, bo_ref, o_ref):
    x = num_ref[0] / (den_ref[0] + 1e-16) + bg_ref[0]
    x = jnp.maximum(x, 0.0)
    o_ref[0] = (
        jnp.dot(x, w_ref[0], preferred_element_type=jnp.float32) + bo_ref[0]
    )


def _tc_in(X, W, A):
    return pl.pallas_call(
        _tc_in_body,
        grid=(2, NB),
        in_specs=[
            pl.BlockSpec((1, BLK, D), lambda s, i: (s, i, 0)),
            pl.BlockSpec((1, D, D), lambda s, i: (s, 0, 0)),
            pl.BlockSpec((1, D, 2), lambda s, i: (s, 0, 0)),
        ],
        out_specs=[
            pl.BlockSpec((1, BLK, D), lambda s, i: (s, i, 0)),
            pl.BlockSpec((1, BLK, 2), lambda s, i: (s, i, 0)),
        ],
        out_shape=[
            jax.ShapeDtypeStruct((2, N, D), jnp.float32),
            jax.ShapeDtypeStruct((2, N, 2), jnp.float32),
        ],
    )(X, W, A)


def _tc_mid(NUM, DEN, B, W, A):
    return pl.pallas_call(
        _tc_mid_body,
        grid=(2, NB),
        in_specs=[
            pl.BlockSpec((1, BLK, D), lambda s, i: (1 - s, i, 0)),
            pl.BlockSpec((1, BLK, 1), lambda s, i: (1 - s, i, 0)),
            pl.BlockSpec((1, 1, D), lambda s, i: (s, 0, 0)),
            pl.BlockSpec((1, D, D), lambda s, i: (s, 0, 0)),
            pl.BlockSpec((1, D, 2), lambda s, i: (s, 0, 0)),
        ],
        out_specs=[
            pl.BlockSpec((1, BLK, D), lambda s, i: (s, i, 0)),
            pl.BlockSpec((1, BLK, 2), lambda s, i: (s, i, 0)),
        ],
        out_shape=[
            jax.ShapeDtypeStruct((2, N, D), jnp.float32),
            jax.ShapeDtypeStruct((2, N, 2), jnp.float32),
        ],
    )(NUM, DEN, B, W, A)


def _tc_out(NUM, DEN, BG, WO, BO):
    return pl.pallas_call(
        _tc_out_body,
        grid=(2, NB),
        in_specs=[
            pl.BlockSpec((1, BLK, D), lambda s, i: (1 - s, i, 0)),
            pl.BlockSpec((1, BLK, 1), lambda s, i: (1 - s, i, 0)),
            pl.BlockSpec((1, 1, D), lambda s, i: (s, 0, 0)),
            pl.BlockSpec((1, D, D), lambda s, i: (s, 0, 0)),
            pl.BlockSpec((1, 1, D), lambda s, i: (s, 0, 0)),
        ],
        out_specs=pl.BlockSpec((1, BLK, D), lambda s, i: (s, i, 0)),
        out_shape=jax.ShapeDtypeStruct((2, N, D), jnp.float32),
    )(NUM, DEN, BG, WO, BO)


def _run_direction(slab, src_hbm, dst_hbm, h_hbm, als_hbm, ald_hbm,
                   num_out, den_out,
                   num_acc, den_fold, als_v, ald_v, src_v, dst_v, dstf_v,
                   h_in, den_up, w_buf, sem, sem_a, sem_b):
    sid = lax.axis_index("s")
    zf = jnp.zeros((L,), jnp.float32)
    iota16 = lax.iota(jnp.int32, L)

    pltpu.sync_copy(als_hbm, als_v)
    pltpu.sync_copy(ald_hbm, ald_v)

    def zrow_e(r, c):
        for cc in range(D // L):
            h_in[r, pl.ds(cc * L, L)] = zf
            den_up[r, pl.ds(cc * L, L)] = zf
        return c
    lax.fori_loop(0, K, zrow_e, 0)

    base = sid * ROWS_A

    def _chunk_row0(k):
        off = jnp.where((k == 7) & (sid != NS - 1), 544, k * K)
        return pl.multiple_of(base + off, 8)

    def _fill_row_idx(r):
        def body(g, c):
            src_v[0, pl.ds(g * L, L)] = r + g * L + iota16
            return c
        lax.fori_loop(0, K // L, body, 0)

    def zcopy_e(k, c):
        r = _chunk_row0(k)
        _fill_row_idx(r)
        pltpu.sync_copy(h_in, num_acc.at[src_v.at[0]])
        return c
    lax.fori_loop(0, 8, zcopy_e, 0)

    @pl.when(sid == 0)
    def _():
        _fill_row_idx(0)
        pltpu.sync_copy(h_in, den_fold.at[src_v.at[0]])

    plsc.subcore_barrier()

    ebase = sid * EPT

    eb0 = pl.multiple_of(ebase, 16)
    pltpu.sync_copy(src_hbm.at[pl.ds(eb0, K)], src_v.at[0])
    pltpu.sync_copy(dst_hbm.at[pl.ds(eb0, K)], dst_v.at[0])

    def chunk(cidx, carry):
        b = cidx & 1
        gd = pltpu.async_copy(h_hbm.at[src_v.at[b]], h_in, sem)
        ebn = pl.multiple_of(
            ebase + jnp.minimum(cidx + 1, NCHUNK - 1) * K, 16)
        ca = pltpu.async_copy(src_hbm.at[pl.ds(ebn, K)], src_v.at[1 - b],
                              sem_a)
        cb = pltpu.async_copy(dst_hbm.at[pl.ds(ebn, K)], dst_v.at[1 - b],
                              sem_b)

        def wgroup(g, c2):
            s16 = src_v[b, pl.ds(g * L, L)]
            d16 = dst_v[b, pl.ds(g * L, L)]
            e16 = plsc.load_gather(als_v, [s16]) + plsc.load_gather(ald_v, [d16])
            e16 = jnp.where(e16 > 0, e16, 0.2 * e16)
            w16 = jnp.exp(e16)
            w_buf[pl.ds(g * L, L)] = w16
            plsc.store_scatter(den_up, [g * L + iota16, d16 & 127], w16)
            dstf_v[pl.ds(g * L, L)] = lax.shift_right_logical(d16, 7)
            return c2
        lax.fori_loop(0, K // L, wgroup, 0)
        gd.wait()
        sb = pltpu.async_copy(den_up, den_fold.at[dstf_v], sem, add=True)

        def sgroup(g, c2):
            for j in range(L):
                e = g * L + j
                sp = plsc.load_gather(w_buf, [jnp.full((L,), e, jnp.int32)])
                for cc in range(D // L):
                    h_in[e, pl.ds(cc * L, L)] = h_in[e, pl.ds(cc * L, L)] * sp
            return c2
        lax.fori_loop(0, K // L, sgroup, 0)
        ca.wait()
        cb.wait()

        sa = pltpu.async_copy(h_in, num_acc.at[dst_v.at[b]], sem_a, add=True)
        sb.wait()

        def clean(g, c2):
            d16 = dst_v[b, pl.ds(g * L, L)]
            plsc.store_scatter(den_up, [g * L + iota16, d16 & 127], zf)
            return c2
        lax.fori_loop(0, K // L, clean, 0)
        sa.wait()
        return carry
    lax.fori_loop(0, NCHUNK, chunk, 0)

    plsc.subcore_barrier()

    def rcopy_e(k, c):
        r = _chunk_row0(k)
        _fill_row_idx(r)
        pltpu.async_copy(num_acc.at[src_v.at[0]], h_in, sem).wait()
        pltpu.sync_copy(h_in, num_out.at[slab, pl.ds(r, K)])
        return c
    lax.fori_loop(0, 8, rcopy_e, 0)

    @pl.when(sid == 0)
    def _():
        _fill_row_idx(0)
        pltpu.async_copy(den_fold.at[src_v.at[0]], h_in, sem).wait()
        pltpu.sync_copy(h_in, den_out.at[slab])
    return


@functools.partial(
    pl.kernel,
    out_type=(
        jax.ShapeDtypeStruct((2, N, D), jnp.float32),
        jax.ShapeDtypeStruct((2, K, D), jnp.float32),
    ),
    mesh=plsc.VectorSubcoreMesh(
        core_axis_name="c", subcore_axis_name="s", num_cores=NC,
        num_subcores=NS,
    ),
    compiler_params=pltpu.CompilerParams(needs_layout_passes=False),
    scratch_types=[
        pltpu.VMEM_SHARED((N, D), jnp.float32),
        pltpu.VMEM_SHARED((K, D), jnp.float32),
        pltpu.VMEM((N,), jnp.float32),
        pltpu.VMEM((N,), jnp.float32),
        pltpu.VMEM((2, K), jnp.int32),
        pltpu.VMEM((2, K), jnp.int32),
        pltpu.VMEM((K,), jnp.int32),
        pltpu.VMEM((K, D), jnp.float32),
        pltpu.VMEM((K, D), jnp.float32),
        pltpu.VMEM((K,), jnp.float32),
        pltpu.SemaphoreType.DMA,
        pltpu.SemaphoreType.DMA,
        pltpu.SemaphoreType.DMA,
    ],
)
def _sc_edge_kernel(src_ui, dst_ui, src_iu, dst_iu, h_u, h_i,
                    als_ui, ald_ui, als_iu, ald_iu,
                    num_out, den_out,
                    num_acc, den_fold, als_v, ald_v, src_v, dst_v, dstf_v,
                    h_in, den_up, w_buf, sem, sem_a, sem_b):
    _sc_body(src_ui, dst_ui, src_iu, dst_iu, h_u, h_i,
             als_ui, ald_ui, als_iu, ald_iu,
             num_out, den_out,
             num_acc, den_fold, als_v, ald_v, src_v, dst_v, dstf_v,
             h_in, den_up, w_buf, sem, sem_a, sem_b)


def _sc_body(src_ui, dst_ui, src_iu, dst_iu, h_u, h_i,
             als_ui, ald_ui, als_iu, ald_iu,
             num_out, den_out,
             num_acc, den_fold, als_v, ald_v, src_v, dst_v, dstf_v,
             h_in, den_up, w_buf, sem, sem_a, sem_b):
    cid = lax.axis_index("c")

    @pl.when(cid == 0)
    def _():
        _run_direction(0, src_ui, dst_ui, h_u, als_ui, ald_ui,
                       num_out, den_out, num_acc, den_fold, als_v, ald_v,
                       src_v, dst_v, dstf_v, h_in, den_up, w_buf, sem, sem_a, sem_b)

    @pl.when(cid == 1)
    def _():
        _run_direction(1, src_iu, dst_iu, h_i, als_iu, ald_iu,
                       num_out, den_out, num_acc, den_fold, als_v, ald_v,
                       src_v, dst_v, dstf_v, h_in, den_up, w_buf, sem, sem_a, sem_b)


def _sc_pass(src_ui, dst_ui, src_iu, dst_iu, H, AL):
    return _sc_edge_kernel(
        src_ui, dst_ui, src_iu, dst_iu, H[0], H[1],
        AL[0, :, 0], AL[1, :, 1], AL[1, :, 0], AL[0, :, 1])


def _den_cols(den):
    return den.reshape(2, K * D)[:, :N, None]


def kernel(x_user, x_item, edge_index_ui, edge_index_iu,
           W_src_ui1, W_dst_ui1, a_src_ui1, a_dst_ui1, b_ui1,
           W_src_iu1, W_dst_iu1, a_src_iu1, a_dst_iu1, b_iu1,
           W_src_ui2, W_dst_ui2, a_src_ui2, a_dst_ui2, b_ui2,
           W_src_iu2, W_dst_iu2, a_src_iu2, a_dst_iu2, b_iu2,
           W_out_user, b_out_user, W_out_item, b_out_item):
    X = jnp.stack([x_user, x_item])
    src_ui, dst_ui = edge_index_ui[0], edge_index_ui[1]
    src_iu, dst_iu = edge_index_iu[0], edge_index_iu[1]

    W1 = jnp.stack([W_src_ui1, W_src_iu1])
    A1 = jnp.stack([
        jnp.stack([W_src_ui1 @ a_src_ui1, W_dst_iu1 @ a_dst_iu1], axis=1),
        jnp.stack([W_src_iu1 @ a_src_iu1, W_dst_ui1 @ a_dst_ui1], axis=1),
    ])
    H1, AL1 = _tc_in(X, W1, A1)
    num1, den1 = _sc_pass(src_ui, dst_ui, src_iu, dst_iu, H1, AL1)

    B1 = jnp.stack([b_iu1, b_ui1])[:, None, :]
    W2 = jnp.stack([W_src_ui2, W_src_iu2])
    A2 = jnp.stack([
        jnp.stack([W_src_ui2 @ a_src_ui2, W_dst_iu2 @ a_dst_iu2], axis=1),
        jnp.stack([W_src_iu2 @ a_src_iu2, W_dst_ui2 @ a_dst_ui2], axis=1),
    ])
    H2, AL2 = _tc_mid(num1, _den_cols(den1), B1, W2, A2)
    num2, den2 = _sc_pass(src_ui, dst_ui, src_iu, dst_iu, H2, AL2)

    BG = jnp.stack([b_iu2, b_ui2])[:, None, :]
    WO = jnp.stack([W_out_user, W_out_item])
    BO = jnp.stack([b_out_user, b_out_item])[:, None, :]
    OUT = _tc_out(num2, _den_cols(den2), BG, WO, BO)
    return OUT[0], OUT[1]
